# baseline (speedup 1.0000x reference)
import functools

import jax
import jax.numpy as jnp
from jax import lax
from jax.experimental import pallas as pl
from jax.experimental.pallas import tpu as pltpu
from jax.experimental.pallas import tpu_sc as plsc

_L = 16
_NC = 2
_NS = 16
_NW = _NC * _NS
_ROWS, _COLS = 16384, 200
_N = _ROWS * _COLS
_PER_W = _N // _NW
_CHUNK = 12800
_NCHUNK = _PER_W // _CHUNK

_mesh = plsc.VectorSubcoreMesh(core_axis_name="c", subcore_axis_name="s")


@functools.partial(
    pl.kernel,
    out_type=jax.ShapeDtypeStruct((_N,), jnp.float32),
    mesh=_mesh,
    scratch_types=[
        pltpu.VMEM((_L,), jnp.float32),
        pltpu.VMEM((_CHUNK,), jnp.int32),
        pltpu.VMEM((_CHUNK,), jnp.float32),
    ],
)
def _week_lookup(d_hbm, w_hbm, out_hbm, w_v, in_v, out_v):
    wid = lax.axis_index("s") * _NC + lax.axis_index("c")
    base = wid * _PER_W
    pltpu.sync_copy(w_hbm, w_v)
    wv = w_v[:]

    def chunk_body(c, carry):
        off = pl.multiple_of(base + c * _CHUNK, 8)
        pltpu.sync_copy(d_hbm.at[pl.ds(off, _CHUNK)], in_v)

        def vec_body(i, carry2):
            x = in_v[pl.ds(i * _L, _L)] + 4
            t = ((x >> 14) << 2) + (x & 16383)
            q = (t * 37450) >> 18
            dow = t - q * 7
            out_v[pl.ds(i * _L, _L)] = jnp.take_along_axis(wv, dow, axis=0)
            return carry2

        lax.fori_loop(0, _CHUNK // _L, vec_body, 0)
        pltpu.sync_copy(out_v, out_hbm.at[pl.ds(off, _CHUNK)])
        return carry

    lax.fori_loop(0, _NCHUNK, chunk_body, 0)


def kernel(day_numbers, w):
    d = day_numbers.astype(jnp.int32).reshape(-1)
    w16 = jnp.zeros((_L,), jnp.float32).at[:7].set(w.astype(jnp.float32))
    out = _week_lookup(d, w16)
    return out.reshape(_ROWS, _COLS)


# speedup vs baseline: 1.1898x; 1.1898x over previous
import functools

import jax
import jax.numpy as jnp
from jax import lax
from jax.experimental import pallas as pl
from jax.experimental.pallas import tpu as pltpu
from jax.experimental.pallas import tpu_sc as plsc

_L = 16
_NC = 2
_NS = 16
_NW = _NC * _NS
_ROWS, _COLS = 16384, 200
_N = _ROWS * _COLS
_PER_W = _N // _NW
_CHUNK = 12800
_NCHUNK = _PER_W // _CHUNK

_mesh = plsc.VectorSubcoreMesh(core_axis_name="c", subcore_axis_name="s")


@functools.partial(
    pl.kernel,
    out_type=jax.ShapeDtypeStruct((_N,), jnp.float32),
    mesh=_mesh,
    scratch_types=[
        pltpu.VMEM((_L,), jnp.float32),
        pltpu.VMEM((_CHUNK,), jnp.int32),
        pltpu.VMEM((_CHUNK,), jnp.int32),
        pltpu.VMEM((_CHUNK,), jnp.float32),
        pltpu.VMEM((_CHUNK,), jnp.float32),
        pltpu.SemaphoreType.DMA,
        pltpu.SemaphoreType.DMA,
        pltpu.SemaphoreType.DMA,
        pltpu.SemaphoreType.DMA,
    ],
)
def _week_lookup(d_hbm, w_hbm, out_hbm, w_v, in0, in1, out0, out1,
                 si0, si1, so0, so1):
    wid = lax.axis_index("s") * _NC + lax.axis_index("c")
    base = wid * _PER_W
    pltpu.sync_copy(w_hbm, w_v)
    wv = w_v[:]

    in_b, out_b = (in0, in1), (out0, out1)
    sin, sout = (si0, si1), (so0, so1)

    def in_slice(c):
        return d_hbm.at[pl.ds(pl.multiple_of(base + c * _CHUNK, 8), _CHUNK)]

    def out_slice(c):
        return out_hbm.at[pl.ds(pl.multiple_of(base + c * _CHUNK, 8), _CHUNK)]

    cp_in = [None, None]
    cp_out = [None, None]
    cp_in[0] = pltpu.async_copy(in_slice(0), in_b[0], sin[0])
    for c in range(_NCHUNK):
        s = c % 2
        if c + 1 < _NCHUNK:
            cp_in[1 - s] = pltpu.async_copy(
                in_slice(c + 1), in_b[1 - s], sin[1 - s])
        cp_in[s].wait()
        if cp_out[s] is not None:
            cp_out[s].wait()
        in_v, out_v = in_b[s], out_b[s]

        @plsc.parallel_loop(0, _CHUNK // _L, unroll=8)
        def vec_body(i):
            x = in_v[pl.ds(i * _L, _L)]
            t = ((x >> 14) << 2) + (x & 16383)
            q = (t * 37450) >> 18
            dow = t - q * 7
            out_v[pl.ds(i * _L, _L)] = jnp.take_along_axis(wv, dow, axis=0)

        cp_out[s] = pltpu.async_copy(out_b[s], out_slice(c), sout[s])
    cp_out[0].wait()
    cp_out[1].wait()


def kernel(day_numbers, w):
    d = day_numbers.astype(jnp.int32).reshape(-1)
    perm = jnp.array([(k + 4) % 7 for k in range(7)], dtype=jnp.int32)
    w16 = jnp.zeros((_L,), jnp.float32).at[:7].set(
        w.astype(jnp.float32)[perm])
    out = _week_lookup(d, w16)
    return out.reshape(_ROWS, _COLS)


# speedup vs baseline: 2.0018x; 1.6825x over previous
import functools

import jax
import jax.numpy as jnp
from jax import lax
from jax.experimental import pallas as pl
from jax.experimental.pallas import tpu as pltpu
from jax.experimental.pallas import tpu_sc as plsc

_L = 16
_NC = 2
_NS = 16
_NW = _NC * _NS
_ROWS, _COLS = 16384, 200
_RPW = _ROWS // _NW
_CROWS = 64
_NCHUNK = _RPW // _CROWS
_COFFS = tuple(range(0, _COLS - _L, _L)) + (_COLS - _L,)

_mesh = plsc.VectorSubcoreMesh(core_axis_name="c", subcore_axis_name="s")


@functools.partial(
    pl.kernel,
    out_type=jax.ShapeDtypeStruct((_ROWS, _COLS), jnp.float32),
    mesh=_mesh,
    scratch_types=[
        pltpu.VMEM((_L,), jnp.float32),
        pltpu.VMEM((_CROWS, _COLS), jnp.int32),
        pltpu.VMEM((_CROWS, _COLS), jnp.int32),
        pltpu.VMEM((_CROWS, _COLS), jnp.float32),
        pltpu.VMEM((_CROWS, _COLS), jnp.float32),
        pltpu.SemaphoreType.DMA,
        pltpu.SemaphoreType.DMA,
        pltpu.SemaphoreType.DMA,
        pltpu.SemaphoreType.DMA,
    ],
)
def _week_lookup(d_hbm, w_hbm, out_hbm, w_v, in0, in1, out0, out1,
                 si0, si1, so0, so1):
    wid = lax.axis_index("s") * _NC + lax.axis_index("c")
    base = wid * _RPW
    pltpu.sync_copy(w_hbm, w_v)
    wv = w_v[:]

    in_b, out_b = (in0, in1), (out0, out1)
    sin, sout = (si0, si1), (so0, so1)

    def in_slice(c):
        return d_hbm.at[pl.ds(pl.multiple_of(base + c * _CROWS, 8), _CROWS), :]

    def out_slice(c):
        return out_hbm.at[pl.ds(pl.multiple_of(base + c * _CROWS, 8), _CROWS), :]

    cp_in = [None, None]
    cp_out = [None, None]
    cp_in[0] = pltpu.async_copy(in_slice(0), in_b[0], sin[0])
    for c in range(_NCHUNK):
        s = c % 2
        if c + 1 < _NCHUNK:
            cp_in[1 - s] = pltpu.async_copy(
                in_slice(c + 1), in_b[1 - s], sin[1 - s])
        cp_in[s].wait()
        if cp_out[s] is not None:
            cp_out[s].wait()
        in_v, out_v = in_b[s], out_b[s]

        @plsc.parallel_loop(0, _CROWS, unroll=2)
        def row_body(r):
            for coff in _COFFS:
                x = in_v[r, pl.ds(coff, _L)]
                t = ((x >> 14) << 2) + (x & 16383)
                q = (t * 37450) >> 18
                dow = t - q * 7
                out_v[r, pl.ds(coff, _L)] = jnp.take_along_axis(
                    wv, dow, axis=0)

        cp_out[s] = pltpu.async_copy(out_b[s], out_slice(c), sout[s])
    cp_out[0].wait()
    cp_out[1].wait()


def kernel(day_numbers, w):
    d = day_numbers.astype(jnp.int32)
    perm = jnp.array([(k + 4) % 7 for k in range(7)], dtype=jnp.int32)
    w16 = jnp.zeros((_L,), jnp.float32).at[:7].set(
        w.astype(jnp.float32)[perm])
    return _week_lookup(d, w16)


# speedup vs baseline: 3.4134x; 1.7052x over previous
import functools

import jax
import jax.numpy as jnp
from jax import lax
from jax.experimental import pallas as pl
from jax.experimental.pallas import tpu as pltpu
from jax.experimental.pallas import tpu_sc as plsc

_L = 16
_NC = 2
_NS = 16
_NW = _NC * _NS
_ROWS, _COLS = 16384, 200
_CPW = _ROWS // _NW
_CROWS = 40
_NCHUNK = _COLS // _CROWS
_NVEC = _CPW // _L
_TBL = 16416

_mesh = plsc.VectorSubcoreMesh(core_axis_name="c", subcore_axis_name="s")


@functools.partial(
    pl.kernel,
    out_type=jax.ShapeDtypeStruct((_COLS, _ROWS), jnp.float32),
    mesh=_mesh,
    compiler_params=pltpu.CompilerParams(
        skip_device_barrier=True, needs_layout_passes=False),
    scratch_types=[
        pltpu.VMEM((_L,), jnp.float32),
        pltpu.VMEM((_TBL,), jnp.float32),
        pltpu.VMEM((_CROWS, _CPW), jnp.int32),
        pltpu.VMEM((_CROWS, _CPW), jnp.int32),
        pltpu.VMEM((_CROWS, _CPW), jnp.float32),
        pltpu.VMEM((_CROWS, _CPW), jnp.float32),
        pltpu.SemaphoreType.DMA,
        pltpu.SemaphoreType.DMA,
        pltpu.SemaphoreType.DMA,
        pltpu.SemaphoreType.DMA,
    ],
)
def _week_lookup(d_hbm, w_hbm, out_hbm, w_v, tbl_v, in0, in1, out0, out1,
                 si0, si1, so0, so1):
    wid = lax.axis_index("s") * _NC + lax.axis_index("c")
    col0 = pl.multiple_of(wid * _CPW, _CPW)

    in_b, out_b = (in0, in1), (out0, out1)
    sin, sout = (si0, si1), (so0, so1)

    def in_slice(c):
        return d_hbm.at[pl.ds(c * _CROWS, _CROWS), pl.ds(col0, _CPW)]

    def out_slice(c):
        return out_hbm.at[pl.ds(c * _CROWS, _CROWS), pl.ds(col0, _CPW)]

    cp_in = [None, None]
    cp_out = [None, None]
    cp_in[0] = pltpu.async_copy(in_slice(0), in_b[0], sin[0])

    pltpu.sync_copy(w_hbm, w_v)
    wv = w_v[:]
    iota4 = lax.iota(jnp.int32, _L) + 4

    @plsc.parallel_loop(0, _TBL // _L)
    def tbl_body(i):
        u = i * _L + iota4
        dow = u - ((u * 37450) >> 18) * 7
        tbl_v[pl.ds(i * _L, _L)] = jnp.take_along_axis(wv, dow, axis=0)

    for c in range(_NCHUNK):
        s = c % 2
        if c + 1 < _NCHUNK:
            cp_in[1 - s] = pltpu.async_copy(
                in_slice(c + 1), in_b[1 - s], sin[1 - s])
        cp_in[s].wait()
        if cp_out[s] is not None:
            cp_out[s].wait()
        in_v, out_v = in_b[s], out_b[s]

        @plsc.parallel_loop(0, _CROWS)
        def row_body(r):
            for v in range(_NVEC):
                x = in_v[r, pl.ds(v * _L, _L)]
                t = ((x >> 14) << 2) + (x & 16383)
                out_v[r, pl.ds(v * _L, _L)] = plsc.load_gather(tbl_v, [t])

        cp_out[s] = pltpu.async_copy(out_b[s], out_slice(c), sout[s])
    cp_out[0].wait()
    cp_out[1].wait()


def kernel(day_numbers, w):
    d = day_numbers.astype(jnp.int32).T
    w16 = jnp.pad(w.astype(jnp.float32), (0, _L - 7))
    return _week_lookup(d, w16).T
